# initial kernel scaffold (unmeasured)
import jax
import jax.numpy as jnp
from jax import lax
from jax.experimental import pallas as pl
from jax.experimental.pallas import tpu as pltpu

N_DEV = 4
B = 2
S = 512
W = 128
HQ = 8
DH = 64
HD = HQ * DH
E = 768
S_EXT = S + 2 * W
S_GLOBAL = N_DEV * S


def kernel(x, Wq, K_ext, V_ext, Wo):
    K2 = K_ext.reshape(B, S, HD)
    V2 = V_ext.reshape(B, S, HD)

    def body(x_ref, wq_ref, k_ref, v_ref, wo_ref, out_ref,
             keff, veff, send_sems, recv_sems):
        my = lax.axis_index("i")
        left = lax.rem(my + N_DEV - 1, N_DEV)
        right = lax.rem(my + 1, N_DEV)

        keff[:, W:W + S, :] = k_ref[...].astype(jnp.bfloat16)
        veff[:, W:W + S, :] = v_ref[...].astype(jnp.bfloat16)

        barrier = pltpu.get_barrier_semaphore()
        for nbr in (left, right):
            pl.semaphore_signal(
                barrier, inc=1,
                device_id=(nbr,), device_id_type=pl.DeviceIdType.MESH,
            )
        pl.semaphore_wait(barrier, 2)

        rdmas = []
        for idx, (buf, src_lo, dst_lo, tgt) in enumerate([
            (keff, S, 0, right),
            (keff, W, S + W, left),
            (veff, S, 0, right),
            (veff, W, S + W, left),
        ]):
            rdma = pltpu.make_async_remote_copy(
                src_ref=buf.at[:, pl.ds(src_lo, W), :],
                dst_ref=buf.at[:, pl.ds(dst_lo, W), :],
                send_sem=send_sems.at[idx],
                recv_sem=recv_sems.at[idx],
                device_id=(tgt,),
                device_id_type=pl.DeviceIdType.MESH,
            )
            rdma.start()
            rdmas.append(rdma)

        wq_b = wq_ref[...].astype(jnp.bfloat16)
        wo_b = wo_ref[...].astype(jnp.bfloat16)
        q_all = []
        for b in range(B):
            xb = x_ref[b].astype(jnp.bfloat16)
            qb = jax.lax.dot_general(
                xb, wq_b, (((1,), (0,)), ((), ())),
                preferred_element_type=jnp.float32,
            )
            q_all.append(qb.astype(jnp.bfloat16))

        qpos = my * S + lax.broadcasted_iota(jnp.int32, (S, S_EXT), 0)
        kpos = my * S - W + lax.broadcasted_iota(jnp.int32, (S, S_EXT), 1)
        valid = (jnp.abs(qpos - kpos) <= W) & (kpos >= 0) & (kpos < S_GLOBAL)
        bias = jnp.where(valid, 0.0, -1e9).astype(jnp.float32)

        for r in rdmas:
            r.wait()

        for b in range(B):
            kb = keff[b]
            vb = veff[b]
            ctx_heads = []
            for h in range(HQ):
                qh = q_all[b][:, h * DH:(h + 1) * DH]
                kh = kb[:, h * DH:(h + 1) * DH]
                vh = vb[:, h * DH:(h + 1) * DH]
                s = jax.lax.dot_general(
                    qh, kh, (((1,), (1,)), ((), ())),
                    preferred_element_type=jnp.float32,
                )
                s = s * 0.125 + bias
                m = jnp.max(s, axis=-1, keepdims=True)
                e = jnp.exp(s - m)
                w = e / jnp.sum(e, axis=-1, keepdims=True)
                ctx_h = jax.lax.dot_general(
                    w.astype(jnp.bfloat16), vh, (((1,), (0,)), ((), ())),
                    preferred_element_type=jnp.float32,
                )
                ctx_heads.append(ctx_h)
            ctx_b = jnp.concatenate(ctx_heads, axis=1).astype(jnp.bfloat16)
            out_ref[b] = jax.lax.dot_general(
                ctx_b, wo_b, (((1,), (0,)), ((), ())),
                preferred_element_type=jnp.float32,
            )

    return pl.pallas_call(
        body,
        out_shape=jax.ShapeDtypeStruct((B, S, E), jnp.float32),
        in_specs=[pl.BlockSpec(memory_space=pltpu.VMEM)] * 5,
        out_specs=pl.BlockSpec(memory_space=pltpu.VMEM),
        scratch_shapes=[
            pltpu.VMEM((B, S_EXT, HD), jnp.bfloat16),
            pltpu.VMEM((B, S_EXT, HD), jnp.bfloat16),
            pltpu.SemaphoreType.DMA((4,)),
            pltpu.SemaphoreType.DMA((4,)),
        ],
        compiler_params=pltpu.CompilerParams(collective_id=0),
    )(x, Wq, K2, V2, Wo)


# baseline (device time: 33377 ns/iter reference)
import os

import jax
import jax.numpy as jnp
from jax import lax
from jax.experimental import pallas as pl
from jax.experimental.pallas import tpu as pltpu

_DEBUG_NO_RDMA = os.environ.get("SCBAND_DEBUG_NO_RDMA") == "1"
_DEBUG_NO_BARRIER = os.environ.get("SCBAND_DEBUG_NO_BARRIER") == "1"

N_DEV = 4
B = 2
S = 512
W = 128
HQ = 8
DH = 64
HD = HQ * DH
E = 768
S_EXT = S + 2 * W
S_GLOBAL = N_DEV * S


def kernel(x, Wq, K_ext, V_ext, Wo):
    K2 = K_ext.reshape(B, S, HD)
    V2 = V_ext.reshape(B, S, HD)

    def body(x_ref, wq_ref, k_ref, v_ref, wo_ref, out_ref,
             keff, veff, send_sems, recv_sems):
        my = lax.axis_index("i")
        left = lax.rem(my + N_DEV - 1, N_DEV)
        right = lax.rem(my + 1, N_DEV)

        keff[:, W:W + S, :] = k_ref[...].astype(jnp.bfloat16)
        veff[:, W:W + S, :] = v_ref[...].astype(jnp.bfloat16)

        if not _DEBUG_NO_BARRIER:
            barrier = pltpu.get_barrier_semaphore()
            for nbr in (left, right):
                pl.semaphore_signal(
                    barrier, inc=1,
                    device_id=(nbr,), device_id_type=pl.DeviceIdType.MESH,
                )
            pl.semaphore_wait(barrier, 2)

        flows = [] if _DEBUG_NO_RDMA else [
            (buf, b, src_lo, dst_lo, tgt)
            for (buf, src_lo, dst_lo, tgt) in [
                (keff, S, 0, right),
                (keff, W, S + W, left),
                (veff, S, 0, right),
                (veff, W, S + W, left),
            ]
            for b in range(B)
        ]
        rdmas = []
        for idx, (buf, b, src_lo, dst_lo, tgt) in enumerate(flows):
            rdma = pltpu.make_async_remote_copy(
                src_ref=buf.at[b, pl.ds(src_lo, W), :],
                dst_ref=buf.at[b, pl.ds(dst_lo, W), :],
                send_sem=send_sems.at[idx],
                recv_sem=recv_sems.at[idx],
                device_id=(tgt,),
                device_id_type=pl.DeviceIdType.MESH,
            )
            rdma.start()
            rdmas.append(rdma)

        wq_b = wq_ref[...].astype(jnp.bfloat16)
        wo_b = wo_ref[...].astype(jnp.bfloat16)
        q_all = []
        for b in range(B):
            xb = x_ref[b].astype(jnp.bfloat16)
            qb = jax.lax.dot_general(
                xb, wq_b, (((1,), (0,)), ((), ())),
                preferred_element_type=jnp.float32,
            )
            q_all.append(qb.astype(jnp.bfloat16))

        qpos = my * S + lax.broadcasted_iota(jnp.int32, (S, S_EXT), 0)
        kpos = my * S - W + lax.broadcasted_iota(jnp.int32, (S, S_EXT), 1)
        valid = (jnp.abs(qpos - kpos) <= W) & (kpos >= 0) & (kpos < S_GLOBAL)
        bias = jnp.where(valid, 0.0, -1e9).astype(jnp.float32)

        for r in rdmas:
            r.wait()

        for b in range(B):
            kb = keff[b]
            vb = veff[b]
            ctx_heads = []
            for h in range(HQ):
                qh = q_all[b][:, h * DH:(h + 1) * DH]
                kh = kb[:, h * DH:(h + 1) * DH]
                vh = vb[:, h * DH:(h + 1) * DH]
                s = jax.lax.dot_general(
                    qh, kh, (((1,), (1,)), ((), ())),
                    preferred_element_type=jnp.float32,
                )
                s = s * 0.125 + bias
                m = jnp.max(s, axis=-1, keepdims=True)
                e = jnp.exp(s - m)
                w = e / jnp.sum(e, axis=-1, keepdims=True)
                ctx_h = jax.lax.dot_general(
                    w.astype(jnp.bfloat16), vh, (((1,), (0,)), ((), ())),
                    preferred_element_type=jnp.float32,
                )
                ctx_heads.append(ctx_h)
            ctx_b = jnp.concatenate(ctx_heads, axis=1).astype(jnp.bfloat16)
            out_ref[b] = jax.lax.dot_general(
                ctx_b, wo_b, (((1,), (0,)), ((), ())),
                preferred_element_type=jnp.float32,
            )

    return pl.pallas_call(
        body,
        out_shape=jax.ShapeDtypeStruct((B, S, E), jnp.float32),
        in_specs=[pl.BlockSpec(memory_space=pltpu.VMEM)] * 5,
        out_specs=pl.BlockSpec(memory_space=pltpu.VMEM),
        scratch_shapes=[
            pltpu.VMEM((B, S_EXT, HD), jnp.bfloat16),
            pltpu.VMEM((B, S_EXT, HD), jnp.bfloat16),
            pltpu.SemaphoreType.DMA((8,)),
            pltpu.SemaphoreType.DMA((8,)),
        ],
        compiler_params=pltpu.CompilerParams(collective_id=0),
    )(x, Wq, K2, V2, Wo)
